# baseline (speedup 1.0000x reference)
import jax
import jax.numpy as jnp
from jax.experimental import pallas as pl
from jax.experimental.pallas import tpu as pltpu
from jax.experimental.pallas import tpu_sc as plsc

B, S, D, F, E = 2, 2048, 1024, 4096, 8
T = B * S
G = T // E
BT2 = 256
MW = 128
SCW = 128


def _router_kernel(x_ref, wu_ref, bu_ref, idx2_ref, idx4_ref, idxm_ref,
                   meta_ref, q_ref):
    z = jax.lax.dot_general(
        x_ref[...], wu_ref[...],
        (((1,), (0,)), ((), ())), preferred_element_type=jnp.float32)
    u = jax.nn.sigmoid(z + bu_ref[0, 0])
    meta_ref[...] = jnp.broadcast_to(u, (T, MW))
    n = jnp.clip(jnp.ceil(u * E), 1, E)
    n2 = n.reshape(G, E)
    n64 = jnp.repeat(n2, E, axis=1)
    v64 = (jax.lax.broadcasted_iota(jnp.int32, (G, E * E), 1) % E
           ).astype(jnp.float32)
    oh_eq = (n64 == v64 + 1).astype(jnp.float32)
    q64 = jnp.sum((n64 > v64).astype(jnp.float32), axis=0, keepdims=True)
    qs64 = jnp.sum((n64 > v64 + 1).astype(jnp.float32), axis=0, keepdims=True)
    ii = jax.lax.broadcasted_iota(jnp.int32, (G, G), 0)
    jj = jax.lax.broadcasted_iota(jnp.int32, (G, G), 1)
    tri = (jj < ii).astype(jnp.bfloat16)
    cum_eq = jax.lax.dot_general(
        tri, oh_eq.astype(jnp.bfloat16),
        (((1,), (0,)), ((), ())), preferred_element_type=jnp.float32)
    a64 = oh_eq * (qs64 + cum_eq)
    pos = jnp.sum(a64.reshape(G, E, E), axis=2)
    r2 = jax.lax.broadcasted_iota(jnp.int32, (G, E), 1)
    slot = pos.astype(jnp.int32) + r2 * G
    h2 = jax.lax.broadcasted_iota(jnp.int32, (G, 2 * E), 1) % 2
    idx2_ref[...] = jnp.repeat(2 * slot, 2, axis=1) + h2
    c4 = jax.lax.broadcasted_iota(jnp.int32, (G, 4 * E), 1) % 4
    idx4_ref[...] = jnp.repeat(slot, 4, axis=1) + c4 * T
    idxm_ref[...] = slot
    q_ref[...] = q64.astype(jnp.int32)


def _run_router(xb, wub, bu2):
    return pl.pallas_call(
        _router_kernel,
        grid=(1,),
        in_specs=[
            pl.BlockSpec((T, D), lambda i: (0, 0)),
            pl.BlockSpec((D, 1), lambda i: (0, 0)),
            pl.BlockSpec((1, 1), lambda i: (0, 0)),
        ],
        out_specs=[
            pl.BlockSpec((G, 2 * E), lambda i: (0, 0)),
            pl.BlockSpec((G, 4 * E), lambda i: (0, 0)),
            pl.BlockSpec((G, E), lambda i: (0, 0)),
            pl.BlockSpec((T, MW), lambda i: (0, 0)),
            pl.BlockSpec((1, E * E), lambda i: (0, 0)),
        ],
        out_shape=[
            jax.ShapeDtypeStruct((G, 2 * E), jnp.int32),
            jax.ShapeDtypeStruct((G, 4 * E), jnp.int32),
            jax.ShapeDtypeStruct((G, E), jnp.int32),
            jax.ShapeDtypeStruct((T, MW), jnp.float32),
            jax.ShapeDtypeStruct((1, E * E), jnp.int32),
        ],
    )(xb, wub, bu2)


def _sc_permute_one(src, idx, scatter):
    N, W = src.shape
    vector_mesh = plsc.VectorSubcoreMesh(
        core_axis_name="core", subcore_axis_name="subcore")

    @pl.kernel(out_type=jax.ShapeDtypeStruct(src.shape, src.dtype),
               mesh=vector_mesh)
    def kperm(s_hbm, i_hbm, d_hbm):
        if scatter:
            def body(s_vmem, i_vmem):
                pltpu.sync_copy(s_vmem, d_hbm.at[i_vmem.at[0]])

            pltpu.emit_pipeline(
                body,
                grid=(N // SCW,),
                in_specs=[
                    pl.BlockSpec((SCW, W), lambda i: (i, 0)),
                    pl.BlockSpec((1, SCW), lambda i: (0, i)),
                ],
                out_specs=[],
                core_axis_name=("core", "subcore"),
                dimension_semantics=(pltpu.PARALLEL,),
            )(s_hbm, i_hbm)
        else:
            def body(i_vmem, o_vmem):
                pltpu.sync_copy(s_hbm.at[i_vmem.at[0]], o_vmem)

            pltpu.emit_pipeline(
                body,
                grid=(N // SCW,),
                in_specs=[pl.BlockSpec((1, SCW), lambda i: (0, i))],
                out_specs=[pl.BlockSpec((SCW, W), lambda i: (i, 0))],
                core_axis_name=("core", "subcore"),
                dimension_semantics=(pltpu.PARALLEL,),
            )(i_hbm, d_hbm)

    return kperm(src, idx)


def _sc_scatter(xb, meta, idx2_flat, idxm_flat):
    x_i32 = jax.lax.bitcast_convert_type(
        xb.reshape(T, D // 2, 2), jnp.int32).reshape(2 * T, D // 4)
    xs2 = _sc_permute_one(x_i32, idx2_flat, True)
    ms = _sc_permute_one(meta, idxm_flat, True)
    xs = jax.lax.bitcast_convert_type(
        xs2.reshape(T, D // 2), jnp.bfloat16).reshape(T, D)
    return xs, ms


def _sparse_ffn_kernel(q_ref, xs_ref, ms_ref, w1_ref, b1_ref, w2_ref, b2_ref,
                       out_ref):
    j = pl.program_id(0)
    p = pl.program_id(1)

    @pl.when(jnp.logical_and(j == 0, p == 0))
    def _():
        for blk in range(4 * T // G):
            out_ref[pl.ds(blk * G, G), :] = jnp.zeros((G, D // 4), jnp.float32)

    w1 = w1_ref[0]
    b1 = b1_ref[0]
    w2 = w2_ref[0]
    b2 = b2_ref[0]
    for r in range(E):
        k = (j + (E - r)) & (E - 1)
        kf = k.astype(jnp.float32)
        q_rk = q_ref[r * E + k]
        nb = (q_rk + BT2 - 1) // BT2

        def body(tb, _, r=r, kf=kf):
            row0 = r * G + tb * BT2
            rows = pl.ds(row0, BT2)
            u_col = ms_ref[rows, 0:1]
            n_col = jnp.clip(jnp.ceil(u_col * E), 1, E)
            c_col = jnp.where(n_col > kf, u_col / (kf + 1.0), 0.0)
            h = jax.lax.dot_general(
                xs_ref[rows, :], w1,
                (((1,), (0,)), ((), ())), preferred_element_type=jnp.float32)
            h = jnp.maximum(h + b1, 0.0)
            hw = (h * c_col).astype(jnp.bfloat16)
            y = jax.lax.dot_general(
                hw, w2,
                (((1,), (0,)), ((), ())), preferred_element_type=jnp.float32)
            y = jnp.where(p == 0, y + c_col * b2, y)
            for c in range(4):
                out_ref[pl.ds(c * T + row0, BT2), :] += (
                    y[:, c * (D // 4):(c + 1) * (D // 4)])
            return 0

        jax.lax.fori_loop(0, nb, body, 0)


def _run_sparse_ffn(q_flat, xs, ms, w1b, b1r, w2b, b2r):
    FH = F // 2
    grid_spec = pltpu.PrefetchScalarGridSpec(
        num_scalar_prefetch=1,
        grid=(E, 2),
        in_specs=[
            pl.BlockSpec((T, D), lambda j, p, q: (0, 0)),
            pl.BlockSpec((T, E), lambda j, p, q: (0, 0)),
            pl.BlockSpec((1, D, FH), lambda j, p, q: (j, 0, p)),
            pl.BlockSpec((1, 1, FH), lambda j, p, q: (j, 0, p)),
            pl.BlockSpec((1, FH, D), lambda j, p, q: (j, p, 0)),
            pl.BlockSpec((1, 1, D), lambda j, p, q: (j, 0, 0)),
        ],
        out_specs=pl.BlockSpec((4 * T, D // 4), lambda j, p, q: (0, 0)),
    )
    return pl.pallas_call(
        _sparse_ffn_kernel,
        grid_spec=grid_spec,
        out_shape=jax.ShapeDtypeStruct((4 * T, D // 4), jnp.float32),
        compiler_params=pltpu.CompilerParams(
            dimension_semantics=("arbitrary", "arbitrary"),
        ),
    )(q_flat, xs, ms[:, :E], w1b, b1r, w2b, b2r)


def _sc_gather(outs4, idx4_flat):
    gathered = _sc_permute_one(outs4, idx4_flat, False)
    return gathered.reshape(T, D)


@jax.jit
def kernel(x, W1, b1, W2, b2, Wu, bu):
    xb = x.reshape(T, D).astype(jnp.bfloat16)
    w1b = W1.astype(jnp.bfloat16)
    w2b = W2.astype(jnp.bfloat16)
    wub = Wu.astype(jnp.bfloat16)
    bu2 = bu.reshape(1, 1)
    b1r = b1.reshape(E, 1, F)
    b2r = b2.reshape(E, 1, D)

    idx2, idx4, idxm, meta, q = _run_router(xb, wub, bu2)
    xs, ms = _sc_scatter(xb, meta, idx2.reshape(1, 2 * T),
                         idxm.reshape(1, T))
    outs4 = _run_sparse_ffn(q.reshape(E * E), xs, ms, w1b, b1r, w2b, b2r)
    out = _sc_gather(outs4, idx4.reshape(1, 4 * T))
    return out.reshape(B, S, D)


# speedup vs baseline: 1.3123x; 1.3123x over previous
import jax
import jax.numpy as jnp
from jax.experimental import pallas as pl
from jax.experimental.pallas import tpu as pltpu
from jax.experimental.pallas import tpu_sc as plsc

B, S, D, F, E = 2, 2048, 1024, 4096, 8
T = B * S
G = T // E
BT2 = 256
MW = 128
SCW = 128
FQ = F // 4
DQ = D // 4


def _router_kernel(x_ref, wu_ref, bu_ref, idxq_ref, idx4_ref, idxm_ref,
                   meta_ref, q_ref):
    z = jax.lax.dot_general(
        x_ref[...].astype(jnp.bfloat16), wu_ref[...],
        (((1,), (0,)), ((), ())), preferred_element_type=jnp.float32)
    u = jax.nn.sigmoid(z + bu_ref[0, 0])
    meta_ref[...] = jnp.broadcast_to(u, (T, MW))
    n = jnp.clip(jnp.ceil(u * E), 1, E)
    n2 = n.reshape(G, E)
    n64 = jnp.repeat(n2, E, axis=1)
    v64 = (jax.lax.broadcasted_iota(jnp.int32, (G, E * E), 1) % E
           ).astype(jnp.float32)
    oh_eq = (n64 == v64 + 1).astype(jnp.float32)
    q64 = jnp.sum((n64 > v64).astype(jnp.float32), axis=0, keepdims=True)
    qs64 = jnp.sum((n64 > v64 + 1).astype(jnp.float32), axis=0, keepdims=True)
    ii = jax.lax.broadcasted_iota(jnp.int32, (G, G), 0)
    jj = jax.lax.broadcasted_iota(jnp.int32, (G, G), 1)
    tri = (jj < ii).astype(jnp.bfloat16)
    cum_eq = jax.lax.dot_general(
        tri, oh_eq.astype(jnp.bfloat16),
        (((1,), (0,)), ((), ())), preferred_element_type=jnp.float32)
    a64 = oh_eq * (qs64 + cum_eq)
    pos = jnp.sum(a64.reshape(G, E, E), axis=2)
    r2 = jax.lax.broadcasted_iota(jnp.int32, (G, E), 1)
    slot = pos.astype(jnp.int32) + r2 * G
    c4 = jax.lax.broadcasted_iota(jnp.int32, (G, 4 * E), 1) % 4
    idxq_ref[...] = jnp.repeat(4 * slot, 4, axis=1) + c4
    idx4_ref[...] = jnp.repeat(slot, 4, axis=1) + c4 * T
    idxm_ref[...] = slot
    q_ref[...] = q64.astype(jnp.int32)


def _run_router(x2, wub, bu2):
    return pl.pallas_call(
        _router_kernel,
        grid=(1,),
        in_specs=[
            pl.BlockSpec((T, D), lambda i: (0, 0)),
            pl.BlockSpec((D, 1), lambda i: (0, 0)),
            pl.BlockSpec((1, 1), lambda i: (0, 0)),
        ],
        out_specs=[
            pl.BlockSpec((G, 4 * E), lambda i: (0, 0)),
            pl.BlockSpec((G, 4 * E), lambda i: (0, 0)),
            pl.BlockSpec((G, E), lambda i: (0, 0)),
            pl.BlockSpec((T, MW), lambda i: (0, 0)),
            pl.BlockSpec((1, E * E), lambda i: (0, 0)),
        ],
        out_shape=[
            jax.ShapeDtypeStruct((G, 4 * E), jnp.int32),
            jax.ShapeDtypeStruct((G, 4 * E), jnp.int32),
            jax.ShapeDtypeStruct((G, E), jnp.int32),
            jax.ShapeDtypeStruct((T, MW), jnp.float32),
            jax.ShapeDtypeStruct((1, E * E), jnp.int32),
        ],
    )(x2, wub, bu2)


def _sc_permute_one(src, idx, scatter):
    N, W = src.shape
    vector_mesh = plsc.VectorSubcoreMesh(
        core_axis_name="core", subcore_axis_name="subcore")

    @pl.kernel(out_type=jax.ShapeDtypeStruct(src.shape, src.dtype),
               mesh=vector_mesh)
    def kperm(s_hbm, i_hbm, d_hbm):
        if scatter:
            def body(s_vmem, i_vmem):
                pltpu.sync_copy(s_vmem, d_hbm.at[i_vmem.at[0]])

            pltpu.emit_pipeline(
                body,
                grid=(N // SCW,),
                in_specs=[
                    pl.BlockSpec((SCW, W), lambda i: (i, 0)),
                    pl.BlockSpec((1, SCW), lambda i: (0, i)),
                ],
                out_specs=[],
                core_axis_name=("core", "subcore"),
                dimension_semantics=(pltpu.PARALLEL,),
            )(s_hbm, i_hbm)
        else:
            def body(i_vmem, o_vmem):
                pltpu.sync_copy(s_hbm.at[i_vmem.at[0]], o_vmem)

            pltpu.emit_pipeline(
                body,
                grid=(N // SCW,),
                in_specs=[pl.BlockSpec((1, SCW), lambda i: (0, i))],
                out_specs=[pl.BlockSpec((SCW, W), lambda i: (i, 0))],
                core_axis_name=("core", "subcore"),
                dimension_semantics=(pltpu.PARALLEL,),
            )(i_hbm, d_hbm)

    return kperm(src, idx)


def _sparse_ffn_kernel(q_ref, xs_ref, ms_ref, w1_ref, b1_ref, w2_ref, b2_ref,
                       out_ref):
    j = pl.program_id(0)
    p = pl.program_id(1)

    @pl.when(jnp.logical_and(j == 0, p == 0))
    def _():
        for blk in range(4 * T // G):
            out_ref[pl.ds(blk * G, G), :] = jnp.zeros((G, DQ), jnp.float32)

    w1 = w1_ref[0].astype(jnp.bfloat16)
    b1 = b1_ref[0]
    w2 = w2_ref[0].astype(jnp.bfloat16)
    b2 = b2_ref[0]
    for r in range(E):
        k = (j + (E - r)) & (E - 1)
        kf = k.astype(jnp.float32)
        q_rk = q_ref[r * E + k]
        nb = (q_rk + BT2 - 1) // BT2

        def body(tb, _, r=r, kf=kf):
            row0 = r * G + tb * BT2
            rows = pl.ds(row0, BT2)
            u_col = ms_ref[rows, 0:1]
            n_col = jnp.clip(jnp.ceil(u_col * E), 1, E)
            c_col = jnp.where(n_col > kf, u_col / (kf + 1.0), 0.0)
            h = jax.lax.dot_general(
                xs_ref[rows, :].astype(jnp.bfloat16), w1,
                (((1,), (0,)), ((), ())), preferred_element_type=jnp.float32)
            h = jnp.maximum(h + b1, 0.0)
            hw = (h * c_col).astype(jnp.bfloat16)
            y = jax.lax.dot_general(
                hw, w2,
                (((1,), (0,)), ((), ())), preferred_element_type=jnp.float32)
            y = jnp.where(p == 0, y + c_col * b2, y)
            for c in range(4):
                out_ref[pl.ds(c * T + row0, BT2), :] += (
                    y[:, c * DQ:(c + 1) * DQ])
            return 0

        jax.lax.fori_loop(0, nb, body, 0)


def _run_sparse_ffn(q_flat, xs, ms, W1, b1r, W2, b2r):
    grid_spec = pltpu.PrefetchScalarGridSpec(
        num_scalar_prefetch=1,
        grid=(E, 4),
        in_specs=[
            pl.BlockSpec((T, D), lambda j, p, q: (0, 0)),
            pl.BlockSpec((T, E), lambda j, p, q: (0, 0)),
            pl.BlockSpec((1, D, FQ), lambda j, p, q: (j, 0, p)),
            pl.BlockSpec((1, 1, FQ), lambda j, p, q: (j, 0, p)),
            pl.BlockSpec((1, FQ, D), lambda j, p, q: (j, p, 0)),
            pl.BlockSpec((1, 1, D), lambda j, p, q: (j, 0, 0)),
        ],
        out_specs=pl.BlockSpec((4 * T, DQ), lambda j, p, q: (0, 0)),
    )
    return pl.pallas_call(
        _sparse_ffn_kernel,
        grid_spec=grid_spec,
        out_shape=jax.ShapeDtypeStruct((4 * T, DQ), jnp.float32),
        compiler_params=pltpu.CompilerParams(
            dimension_semantics=("arbitrary", "arbitrary"),
        ),
    )(q_flat, xs, ms[:, :E], W1, b1r, W2, b2r)


@jax.jit
def kernel(x, W1, b1, W2, b2, Wu, bu):
    x2 = x.reshape(T, D)
    wub = Wu.astype(jnp.bfloat16)
    bu2 = bu.reshape(1, 1)
    b1r = b1.reshape(E, 1, F)
    b2r = b2.reshape(E, 1, D)

    idxq, idx4, idxm, meta, q = _run_router(x2, wub, bu2)
    xs4 = _sc_permute_one(x2.reshape(4 * T, DQ), idxq.reshape(1, 4 * T), True)
    ms = _sc_permute_one(meta, idxm.reshape(1, T), True)
    outs4 = _run_sparse_ffn(q.reshape(E * E), xs4.reshape(T, D), ms,
                            W1, b1r, W2, b2r)
    out = _sc_permute_one(outs4, idx4.reshape(1, 4 * T), False)
    return out.reshape(B, S, D)
